# baseline (speedup 1.0000x reference)
import functools

import jax
import jax.numpy as jnp
from jax import lax
from jax.experimental import pallas as pl
from jax.experimental.pallas import tpu as pltpu
from jax.experimental.pallas import tpu_sc as plsc

N = 10000
E = 320000
D = 128
H = 128
G = 64

NC = 2
NS = 16
NW = NC * NS
EPW = E // NW
CHUNK = 80
NCHUNK = EPW // CHUNK
NBUF = 3
RPT = 632
RPT_LAST = N - (NS - 1) * RPT

@functools.cache
def _make_sc_agg():
    mesh = plsc.VectorSubcoreMesh(core_axis_name="c", subcore_axis_name="s",
                                  num_cores=NC, num_subcores=NS)
    return functools.partial(
        pl.kernel,
        out_type=jax.ShapeDtypeStruct((2, N, H), jnp.float32),
        mesh=mesh,
        scratch_types=[
            pltpu.VMEM((NCHUNK, CHUNK), jnp.int32),
        ] + [pltpu.VMEM((CHUNK, H), jnp.float32) for _ in range(NBUF)]
          + [pltpu.VMEM((CHUNK,), jnp.int32) for _ in range(1 + NBUF)]
          + [pltpu.VMEM_SHARED((N, H), jnp.float32)]
          + [pltpu.SemaphoreType.DMA for _ in range(2 * NBUF + 1)],
    )(_sc_agg_body)


def _sc_agg_body(h_hbm, pk_hbm, out_hbm, pk_idx, b0, b1, b2, src_u,
                 du0, du1, du2, accum, g0, g1, g2, s0, s1, s2, isem):
    c = lax.axis_index("c")
    s = lax.axis_index("s")
    wid = s * NC + c
    bufs = (b0, b1, b2)
    dus = (du0, du1, du2)
    gsems = (g0, g1, g2)
    ssems = (s0, s1, s2)

    row0 = pl.multiple_of(s * RPT, 8)

    def init_copy(fn):
        @pl.when(s < NS - 1)
        def _():
            fn(h_hbm.at[pl.ds(row0, RPT)], accum.at[pl.ds(row0, RPT)])

        @pl.when(s == NS - 1)
        def _():
            fn(h_hbm.at[pl.ds(row0, RPT_LAST)], accum.at[pl.ds(row0, RPT_LAST)])

    init_copy(lambda a, b: pltpu.async_copy(a, b, isem))

    pltpu.sync_copy(pk_hbm.at[wid], pk_idx)

    def unpack(j, du_ref):
        for k in range(CHUNK // 16):
            v = pk_idx[j, pl.ds(16 * k, 16)]
            src_u[pl.ds(16 * k, 16)] = v & jnp.int32(0xFFFF)
            du_ref[pl.ds(16 * k, 16)] = v >> jnp.int32(16)

    unpack(0, dus[0])
    pltpu.async_copy(h_hbm.at[src_u], bufs[0], gsems[0])

    init_copy(lambda a, b: pltpu.make_async_copy(a, b, isem).wait())
    plsc.subcore_barrier()

    def body(i, carry):
        for b in range(NBUF):
            @pl.when(lax.rem(i, NBUF) == b)
            def _(b=b):
                b2 = (b + 1) % NBUF
                pltpu.make_async_copy(h_hbm.at[src_u], bufs[b],
                                      gsems[b]).wait()
                pltpu.async_copy(bufs[b], accum.at[dus[b]], ssems[b],
                                 add=True)

                @pl.when(i >= NBUF - 1)
                def _():
                    pltpu.make_async_copy(bufs[b2], accum.at[dus[b2]],
                                          ssems[b2]).wait()

                @pl.when(i + 1 < NCHUNK)
                def _():
                    unpack(i + 1, dus[b2])
                    pltpu.async_copy(h_hbm.at[src_u], bufs[b2], gsems[b2])
        return carry

    lax.fori_loop(0, NCHUNK, body, 0)
    for i in (NCHUNK - 2, NCHUNK - 1):
        b = i % NBUF
        pltpu.make_async_copy(bufs[b], accum.at[dus[b]], ssems[b]).wait()
    plsc.subcore_barrier()

    @pl.when(s < NS - 1)
    def _():
        pltpu.sync_copy(accum.at[pl.ds(row0, RPT)],
                        out_hbm.at[c, pl.ds(row0, RPT)])

    @pl.when(s == NS - 1)
    def _():
        pltpu.sync_copy(accum.at[pl.ds(row0, RPT_LAST)],
                        out_hbm.at[c, pl.ds(row0, RPT_LAST)])


def _tc_body(parts_ref, h_ref, W1_ref, b1_ref, W2_ref, b2_ref, g_ref, be_ref,
             P_ref, y_ref, pool_ref):
    m = parts_ref[0] + parts_ref[1] - h_ref[...]
    t = jnp.dot(m, W1_ref[...], preferred_element_type=jnp.float32) + b1_ref[...]
    t = jnp.maximum(t, 0.0)
    t = jnp.dot(t, W2_ref[...], preferred_element_type=jnp.float32) + b2_ref[...]
    t = jnp.maximum(t, 0.0)
    mu = jnp.sum(t, axis=0, keepdims=True) * (1.0 / N)
    d = t - mu
    var = jnp.sum(d * d, axis=0, keepdims=True) * (1.0 / N)
    y = d * lax.rsqrt(var + 1e-5) * g_ref[...] + be_ref[...]
    y_ref[...] = y
    pool_ref[...] = lax.dot_general(
        P_ref[...], y, (((0,), (0,)), ((), ())),
        preferred_element_type=jnp.float32,
        precision=lax.Precision.HIGHEST)


_tc_dense = pl.pallas_call(
    _tc_body,
    out_shape=[
        jax.ShapeDtypeStruct((N, H), jnp.float32),
        jax.ShapeDtypeStruct((G, H), jnp.float32),
    ],
)


def kernel(x, edge_index, batch,
           W1_0, b1_0, W2_0, b2_0, g_0, be_0,
           W1_1, b1_1, W2_1, b2_1, g_1, be_1,
           W1_2, b1_2, W2_2, b2_2, g_2, be_2):
    packed = ((edge_index[1] << 16) | edge_index[0]).reshape(NW, NCHUNK, CHUNK)
    P = (batch[:, None] == jnp.arange(G, dtype=batch.dtype)[None, :]).astype(
        jnp.float32)
    plist = [(W1_0, b1_0, W2_0, b2_0, g_0, be_0),
             (W1_1, b1_1, W2_1, b2_1, g_1, be_1),
             (W1_2, b1_2, W2_2, b2_2, g_2, be_2)]
    h = x
    pools = []
    for (W1, b1, W2, b2, g, be) in plist:
        parts = _make_sc_agg()(h, packed)
        h, pool = _tc_dense(parts, h, W1,
                            b1.reshape(1, H), W2, b2.reshape(1, H),
                            g.reshape(1, H), be.reshape(1, H), P)
        pools.append(pool)
    return jnp.concatenate(pools, axis=1)


# speedup vs baseline: 1.3296x; 1.3296x over previous
import functools

import jax
import jax.numpy as jnp
from jax import lax
from jax.experimental import pallas as pl
from jax.experimental.pallas import tpu as pltpu
from jax.experimental.pallas import tpu_sc as plsc

N = 10000
E = 320000
D = 128
H = 128
G = 64

NC = 2
NS = 16
NW = NC * NS
EPW = E // NW
CHUNK = 125
NCHUNK = EPW // CHUNK
SEC = 40
NSEC = NCHUNK // SEC
NBUF = 2
NG = SEC // NBUF
RPT = 632
RPT_LAST = N - (NS - 1) * RPT

@functools.cache
def _make_sc_agg():
    mesh = plsc.VectorSubcoreMesh(core_axis_name="c", subcore_axis_name="s",
                                  num_cores=NC, num_subcores=NS)
    return functools.partial(
        pl.kernel,
        out_type=jax.ShapeDtypeStruct((2, N, H), jnp.float32),
        mesh=mesh,
        scratch_types=[
            pltpu.VMEM((SEC, CHUNK), jnp.int32),
            pltpu.VMEM((SEC, CHUNK), jnp.int32),
        ] + [pltpu.VMEM((CHUNK, H), jnp.float32) for _ in range(NBUF)]
          + [pltpu.VMEM_SHARED((N, H), jnp.float32)]
          + [pltpu.SemaphoreType.DMA for _ in range(NBUF)],
    )(_sc_agg_body)


def _sc_agg_body(h_hbm, src_hbm, dst_hbm, out_hbm, src_idx, dst_idx,
                 b0, b1, accum, s0, s1):
    c = lax.axis_index("c")
    s = lax.axis_index("s")
    wid = s * NC + c
    bufs = (b0, b1)
    sems = (s0, s1)

    row0 = pl.multiple_of(s * RPT, 8)

    @pl.when(s < NS - 1)
    def _():
        pltpu.sync_copy(h_hbm.at[pl.ds(row0, RPT)], accum.at[pl.ds(row0, RPT)])

    @pl.when(s == NS - 1)
    def _():
        pltpu.sync_copy(h_hbm.at[pl.ds(row0, RPT_LAST)],
                        accum.at[pl.ds(row0, RPT_LAST)])

    plsc.subcore_barrier()

    def sec_body(sec, carry):
        sec0 = pl.multiple_of(sec * SEC, 8)
        pltpu.sync_copy(src_hbm.at[wid, pl.ds(sec0, SEC)], src_idx)
        pltpu.sync_copy(dst_hbm.at[wid, pl.ds(sec0, SEC)], dst_idx)
        for b in range(NBUF):
            pltpu.async_copy(h_hbm.at[src_idx.at[b]], bufs[b], sems[b])

        def body(g, c2):
            for b in range(NBUF):
                i = g * NBUF + b
                pltpu.make_async_copy(h_hbm.at[src_idx.at[i]], bufs[b],
                                      sems[b]).wait()
                pltpu.sync_copy(bufs[b], accum.at[dst_idx.at[i]], add=True)
                pltpu.async_copy(h_hbm.at[src_idx.at[i + NBUF]], bufs[b],
                                 sems[b])
            return c2

        lax.fori_loop(0, NG - 1, body, 0)
        for b in range(NBUF):
            i = (NG - 1) * NBUF + b
            pltpu.make_async_copy(h_hbm.at[src_idx.at[i]], bufs[b],
                                  sems[b]).wait()
            pltpu.sync_copy(bufs[b], accum.at[dst_idx.at[i]], add=True)
        return carry

    lax.fori_loop(0, NSEC, sec_body, 0)
    plsc.subcore_barrier()

    @pl.when(s < NS - 1)
    def _():
        pltpu.sync_copy(accum.at[pl.ds(row0, RPT)],
                        out_hbm.at[c, pl.ds(row0, RPT)])

    @pl.when(s == NS - 1)
    def _():
        pltpu.sync_copy(accum.at[pl.ds(row0, RPT_LAST)],
                        out_hbm.at[c, pl.ds(row0, RPT_LAST)])


def _tc_body(parts_ref, h_ref, W1_ref, b1_ref, W2_ref, b2_ref, g_ref, be_ref,
             P_ref, y_ref, pool_ref):
    m = parts_ref[0] + parts_ref[1] - h_ref[...]
    t = jnp.dot(m, W1_ref[...], preferred_element_type=jnp.float32) + b1_ref[...]
    t = jnp.maximum(t, 0.0)
    t = jnp.dot(t, W2_ref[...], preferred_element_type=jnp.float32) + b2_ref[...]
    t = jnp.maximum(t, 0.0)
    mu = jnp.sum(t, axis=0, keepdims=True) * (1.0 / N)
    d = t - mu
    var = jnp.sum(d * d, axis=0, keepdims=True) * (1.0 / N)
    y = d * lax.rsqrt(var + 1e-5) * g_ref[...] + be_ref[...]
    y_ref[...] = y
    pool_ref[...] = lax.dot_general(
        P_ref[...], y, (((0,), (0,)), ((), ())),
        preferred_element_type=jnp.float32,
        precision=lax.Precision.HIGHEST)


_tc_dense = pl.pallas_call(
    _tc_body,
    out_shape=[
        jax.ShapeDtypeStruct((N, H), jnp.float32),
        jax.ShapeDtypeStruct((G, H), jnp.float32),
    ],
)


def kernel(x, edge_index, batch,
           W1_0, b1_0, W2_0, b2_0, g_0, be_0,
           W1_1, b1_1, W2_1, b2_1, g_1, be_1,
           W1_2, b1_2, W2_2, b2_2, g_2, be_2):
    src = edge_index[0].reshape(NW, NCHUNK, CHUNK)
    dst = edge_index[1].reshape(NW, NCHUNK, CHUNK)
    P = (batch[:, None] == jnp.arange(G, dtype=batch.dtype)[None, :]).astype(
        jnp.float32)
    plist = [(W1_0, b1_0, W2_0, b2_0, g_0, be_0),
             (W1_1, b1_1, W2_1, b2_1, g_1, be_1),
             (W1_2, b1_2, W2_2, b2_2, g_2, be_2)]
    h = x
    pools = []
    for (W1, b1, W2, b2, g, be) in plist:
        parts = _make_sc_agg()(h, src, dst)
        h, pool = _tc_dense(parts, h, W1,
                            b1.reshape(1, H), W2, b2.reshape(1, H),
                            g.reshape(1, H), be.reshape(1, H), P)
        pools.append(pool)
    return jnp.concatenate(pools, axis=1)


# speedup vs baseline: 1.3592x; 1.0223x over previous
import functools

import jax
import jax.numpy as jnp
from jax import lax
from jax.experimental import pallas as pl
from jax.experimental.pallas import tpu as pltpu
from jax.experimental.pallas import tpu_sc as plsc

N = 10000
E = 320000
D = 128
H = 128
G = 64

NC = 2
NS = 16
NW = NC * NS
EPW = E // NW
CHUNK = 125
NCHUNK = EPW // CHUNK
SEC = 40
NSEC = NCHUNK // SEC
NBUF = 2
NG = SEC // NBUF
RPT = 632
RPT_LAST = N - (NS - 1) * RPT

@functools.cache
def _make_sc_agg():
    mesh = plsc.VectorSubcoreMesh(core_axis_name="c", subcore_axis_name="s",
                                  num_cores=NC, num_subcores=NS)
    return functools.partial(
        pl.kernel,
        out_type=jax.ShapeDtypeStruct((2, N, H), jnp.float32),
        mesh=mesh,
        scratch_types=[
            pltpu.VMEM((SEC, CHUNK), jnp.int32),
            pltpu.VMEM((SEC, CHUNK), jnp.int32),
        ] + [pltpu.VMEM((CHUNK, H), jnp.float32) for _ in range(NBUF)]
          + [pltpu.VMEM_SHARED((N, H), jnp.float32)]
          + [pltpu.SemaphoreType.DMA for _ in range(NBUF + 1)],
    )(_sc_agg_body)


def _sc_agg_body(h_hbm, src_hbm, dst_hbm, out_hbm, src_idx, dst_idx,
                 b0, b1, accum, s0, s1, isem):
    c = lax.axis_index("c")
    s = lax.axis_index("s")
    wid = s * NC + c
    bufs = (b0, b1)
    sems = (s0, s1)

    row0 = pl.multiple_of(s * RPT, 8)

    @pl.when(s < NS - 1)
    def _():
        pltpu.async_copy(h_hbm.at[pl.ds(row0, RPT)],
                         accum.at[pl.ds(row0, RPT)], isem)

    @pl.when(s == NS - 1)
    def _():
        pltpu.async_copy(h_hbm.at[pl.ds(row0, RPT_LAST)],
                         accum.at[pl.ds(row0, RPT_LAST)], isem)

    def sec_body(sec, carry):
        sec0 = pl.multiple_of(sec * SEC, 8)
        pltpu.sync_copy(src_hbm.at[wid, pl.ds(sec0, SEC)], src_idx)
        pltpu.sync_copy(dst_hbm.at[wid, pl.ds(sec0, SEC)], dst_idx)
        for b in range(NBUF):
            pltpu.async_copy(h_hbm.at[src_idx.at[b]], bufs[b], sems[b])

        @pl.when(sec == 0)
        def _():
            @pl.when(s < NS - 1)
            def _():
                pltpu.make_async_copy(h_hbm.at[pl.ds(row0, RPT)],
                                      accum.at[pl.ds(row0, RPT)], isem).wait()

            @pl.when(s == NS - 1)
            def _():
                pltpu.make_async_copy(h_hbm.at[pl.ds(row0, RPT_LAST)],
                                      accum.at[pl.ds(row0, RPT_LAST)],
                                      isem).wait()

            plsc.subcore_barrier()

        def body(g, c2):
            for b in range(NBUF):
                i = g * NBUF + b
                pltpu.make_async_copy(h_hbm.at[src_idx.at[i]], bufs[b],
                                      sems[b]).wait()
                pltpu.sync_copy(bufs[b], accum.at[dst_idx.at[i]], add=True)
                pltpu.async_copy(h_hbm.at[src_idx.at[i + NBUF]], bufs[b],
                                 sems[b])
            return c2

        lax.fori_loop(0, NG - 1, body, 0)
        for b in range(NBUF):
            i = (NG - 1) * NBUF + b
            pltpu.make_async_copy(h_hbm.at[src_idx.at[i]], bufs[b],
                                  sems[b]).wait()
            pltpu.sync_copy(bufs[b], accum.at[dst_idx.at[i]], add=True)
        return carry

    lax.fori_loop(0, NSEC, sec_body, 0)
    plsc.subcore_barrier()

    @pl.when(s < NS - 1)
    def _():
        pltpu.sync_copy(accum.at[pl.ds(row0, RPT)],
                        out_hbm.at[c, pl.ds(row0, RPT)])

    @pl.when(s == NS - 1)
    def _():
        pltpu.sync_copy(accum.at[pl.ds(row0, RPT_LAST)],
                        out_hbm.at[c, pl.ds(row0, RPT_LAST)])


def _tc_body(parts_ref, h_ref, W1_ref, b1_ref, W2_ref, b2_ref, g_ref, be_ref,
             P_ref, y_ref, pool_ref):
    m = parts_ref[0] + parts_ref[1] - h_ref[...]
    t = jnp.dot(m, W1_ref[...], preferred_element_type=jnp.float32) + b1_ref[...]
    t = jnp.maximum(t, 0.0)
    t = jnp.dot(t, W2_ref[...], preferred_element_type=jnp.float32) + b2_ref[...]
    t = jnp.maximum(t, 0.0)
    mu = jnp.sum(t, axis=0, keepdims=True) * (1.0 / N)
    d = t - mu
    var = jnp.sum(d * d, axis=0, keepdims=True) * (1.0 / N)
    y = d * lax.rsqrt(var + 1e-5) * g_ref[...] + be_ref[...]
    y_ref[...] = y
    pool_ref[...] = lax.dot_general(
        P_ref[...], y, (((0,), (0,)), ((), ())),
        preferred_element_type=jnp.float32,
        precision=lax.Precision.HIGHEST)


_tc_dense = pl.pallas_call(
    _tc_body,
    out_shape=[
        jax.ShapeDtypeStruct((N, H), jnp.float32),
        jax.ShapeDtypeStruct((G, H), jnp.float32),
    ],
)


def kernel(x, edge_index, batch,
           W1_0, b1_0, W2_0, b2_0, g_0, be_0,
           W1_1, b1_1, W2_1, b2_1, g_1, be_1,
           W1_2, b1_2, W2_2, b2_2, g_2, be_2):
    src = edge_index[0].reshape(NW, NCHUNK, CHUNK)
    dst = edge_index[1].reshape(NW, NCHUNK, CHUNK)
    P = (batch[:, None] == jnp.arange(G, dtype=batch.dtype)[None, :]).astype(
        jnp.float32)
    plist = [(W1_0, b1_0, W2_0, b2_0, g_0, be_0),
             (W1_1, b1_1, W2_1, b2_1, g_1, be_1),
             (W1_2, b1_2, W2_2, b2_2, g_2, be_2)]
    h = x
    pools = []
    for (W1, b1, W2, b2, g, be) in plist:
        parts = _make_sc_agg()(h, src, dst)
        h, pool = _tc_dense(parts, h, W1,
                            b1.reshape(1, H), W2, b2.reshape(1, H),
                            g.reshape(1, H), be.reshape(1, H), P)
        pools.append(pool)
    return jnp.concatenate(pools, axis=1)
